# baseline (speedup 1.0000x reference)
import jax
import jax.numpy as jnp
from jax.experimental import pallas as pl
from jax.experimental.pallas import tpu as pltpu

N_DEST = 4096
N_SRC = 4096
FEAT = 1024
TFEAT = 256
EMB = 64
HID = 128
HEADS = 4

D_BLK = 256
E_BLK = 1024

_MASK = -3e38


def _dot(a, b, dims):
    return jax.lax.dot_general(
        a.astype(jnp.bfloat16), b.astype(jnp.bfloat16), dims,
        preferred_element_type=jnp.float32,
    )


def _mm(a, b):
    return _dot(a, b, (((1,), (0,)), ((), ())))


def _enc_qk_kernel(x_ref, w1_ref, b1_ref, w2_ref, b2_ref,
                   ed_ref, es_ref, aw_ref, aw2_ref,
                   tf_ref, tfb_ref, q_ref, kt_ref):
    @pl.when(pl.program_id(0) == 0)
    def _qk():
        for h in range(HEADS):
            h2 = _mm(ed_ref[...], aw_ref[h])
            q_ref[h] = _mm(h2, aw2_ref[h]).astype(jnp.bfloat16)
            kt_ref[h] = _dot(
                aw_ref[h], es_ref[...], (((0,), (1,)), ((), ())),
            ).astype(jnp.bfloat16)

    h = jnp.maximum(_mm(x_ref[...], w1_ref[...]) + b1_ref[...], 0.0)
    tf = _mm(h, w2_ref[...]) + b2_ref[...]
    tf_ref[...] = tf
    tfb_ref[...] = tf.astype(jnp.bfloat16)


def _enc_qk(x, w1, b1, w2, b2, emb_dest, emb_src, att_w, att_w2):
    return pl.pallas_call(
        _enc_qk_kernel,
        grid=(N_SRC // E_BLK,),
        in_specs=[
            pl.BlockSpec((E_BLK, FEAT), lambda i: (i, 0)),
            pl.BlockSpec((FEAT, 2 * TFEAT), lambda i: (0, 0)),
            pl.BlockSpec((1, 2 * TFEAT), lambda i: (0, 0)),
            pl.BlockSpec((2 * TFEAT, TFEAT), lambda i: (0, 0)),
            pl.BlockSpec((1, TFEAT), lambda i: (0, 0)),
            pl.BlockSpec((N_DEST, EMB), lambda i: (0, 0)),
            pl.BlockSpec((N_SRC, EMB), lambda i: (0, 0)),
            pl.BlockSpec((HEADS, EMB, HID), lambda i: (0, 0, 0)),
            pl.BlockSpec((HEADS, HID, HID), lambda i: (0, 0, 0)),
        ],
        out_specs=[
            pl.BlockSpec((E_BLK, TFEAT), lambda i: (i, 0)),
            pl.BlockSpec((E_BLK, TFEAT), lambda i: (i, 0)),
            pl.BlockSpec((HEADS, N_DEST, HID), lambda i: (0, 0, 0)),
            pl.BlockSpec((HEADS, HID, N_SRC), lambda i: (0, 0, 0)),
        ],
        out_shape=[
            jax.ShapeDtypeStruct((N_SRC, TFEAT), jnp.float32),
            jax.ShapeDtypeStruct((N_SRC, TFEAT), jnp.bfloat16),
            jax.ShapeDtypeStruct((HEADS, N_DEST, HID), jnp.bfloat16),
            jax.ShapeDtypeStruct((HEADS, HID, N_SRC), jnp.bfloat16),
        ],
    )(x, w1, b1, w2, b2, emb_dest, emb_src, att_w, att_w2)


def _attn_dec_kernel(q_ref, kt_ref, bias_ref, tfb_ref,
                     dw1_ref, db1_ref, dw2_ref, db2_ref,
                     out_ref, fh_ref):
    mask_add = jnp.where(bias_ref[...] > 0.0, 0.0, _MASK)
    tf_all = tfb_ref[...]

    out = jnp.zeros((D_BLK, TFEAT), jnp.float32)
    for h in range(HEADS):
        s = _mm(q_ref[h], kt_ref[h])
        t = jnp.maximum(s, 0.2 * s) + mask_add
        m = jnp.max(t, axis=1, keepdims=True)
        p = jnp.exp((t - m).astype(jnp.bfloat16))
        l = jnp.sum(p, axis=1, keepdims=True).astype(jnp.float32)
        hp = _mm(p, tf_all) * (1.0 / l)
        out = out + jnp.where(hp > 0.0, hp, jnp.exp(hp) - 1.0)
    out_ref[...] = out * (1.0 / HEADS)

    r0 = pl.multiple_of(pl.program_id(0) * D_BLK, D_BLK)
    tfb = tfb_ref[pl.ds(r0, D_BLK), :]
    hid = jnp.maximum(_mm(tfb, dw1_ref[...]) + db1_ref[...], 0.0)
    fh_ref[...] = _mm(hid, dw2_ref[...]) + db2_ref[...]


def _attn_dec(q, kt, bias, tfb, dw1, db1, dw2, db2):
    return pl.pallas_call(
        _attn_dec_kernel,
        grid=(N_DEST // D_BLK,),
        in_specs=[
            pl.BlockSpec((HEADS, D_BLK, HID), lambda i: (0, i, 0)),
            pl.BlockSpec((HEADS, HID, N_SRC), lambda i: (0, 0, 0)),
            pl.BlockSpec((D_BLK, N_SRC), lambda i: (i, 0)),
            pl.BlockSpec((N_SRC, TFEAT), lambda i: (0, 0)),
            pl.BlockSpec((TFEAT, 2 * TFEAT), lambda i: (0, 0)),
            pl.BlockSpec((1, 2 * TFEAT), lambda i: (0, 0)),
            pl.BlockSpec((2 * TFEAT, FEAT), lambda i: (0, 0)),
            pl.BlockSpec((1, FEAT), lambda i: (0, 0)),
        ],
        out_specs=[
            pl.BlockSpec((D_BLK, TFEAT), lambda i: (i, 0)),
            pl.BlockSpec((D_BLK, FEAT), lambda i: (i, 0)),
        ],
        out_shape=[
            jax.ShapeDtypeStruct((N_DEST, TFEAT), jnp.float32),
            jax.ShapeDtypeStruct((N_SRC, FEAT), jnp.float32),
        ],
    )(q, kt, bias, tfb, dw1, db1, dw2, db2)


def kernel(bias, emb_dest, emb_src, feature_src, enc_W1, enc_b1, enc_W2,
           enc_b2, dec_W1, dec_b1, dec_W2, dec_b2, att_W, att_W2):
    bf = jnp.bfloat16
    tf, tfb, q, kt = _enc_qk(
        feature_src, enc_W1.astype(bf), enc_b1.reshape(1, -1),
        enc_W2.astype(bf), enc_b2.reshape(1, -1),
        emb_dest, emb_src, att_W.astype(bf), att_W2.astype(bf))
    feature_src_re, feature_hat = _attn_dec(
        q, kt, bias, tfb, dec_W1.astype(bf), dec_b1.reshape(1, -1),
        dec_W2.astype(bf), dec_b2.reshape(1, -1))
    return (feature_src_re, feature_hat, tf)


# speedup vs baseline: 1.0480x; 1.0480x over previous
import jax
import jax.numpy as jnp
from jax.experimental import pallas as pl
from jax.experimental.pallas import tpu as pltpu

N_DEST = 4096
N_SRC = 4096
FEAT = 1024
TFEAT = 256
EMB = 64
HID = 128
HEADS = 4

D_BLK = 256
E_BLK = 1024
N_ENC = N_SRC // E_BLK
N_ATT = N_DEST // D_BLK


def _dot(a, b, dims):
    return jax.lax.dot_general(
        a.astype(jnp.bfloat16), b.astype(jnp.bfloat16), dims,
        preferred_element_type=jnp.float32,
    )


def _mm(a, b):
    return _dot(a, b, (((1,), (0,)), ((), ())))


def _fused_kernel(x_ref, w1_ref, b1_ref, w2_ref, b2_ref,
                  ed_ref, es_ref, aw_ref, aw2_ref,
                  bias_ref, dw1_ref, db1_ref, dw2_ref, db2_ref,
                  tf_ref, out_ref, fh_ref,
                  tfb_ref, q_ref, kt_ref):
    i = pl.program_id(0)

    @pl.when(i == 0)
    def _qk():
        for h in range(HEADS):
            h2 = _mm(ed_ref[...], aw_ref[h])
            q_ref[h] = _mm(h2, aw2_ref[h]).astype(jnp.bfloat16)
            kt_ref[h] = _dot(
                aw_ref[h], es_ref[...], (((0,), (1,)), ((), ())),
            ).astype(jnp.bfloat16)

    @pl.when(i < N_ENC)
    def _encoder():
        h = jnp.maximum(_mm(x_ref[...], w1_ref[...]) + b1_ref[...], 0.0)
        tf = _mm(h, w2_ref[...]) + b2_ref[...]
        tf_ref[...] = tf
        r0 = pl.multiple_of(i * E_BLK, E_BLK)
        tfb_ref[pl.ds(r0, E_BLK), :] = tf.astype(jnp.bfloat16)

    @pl.when(i >= N_ENC)
    def _attn_dec():
        d0 = pl.multiple_of((i - N_ENC) * D_BLK, D_BLK)
        mask_add = jnp.where(bias_ref[...] > 0.0, 0.0, -3e38)
        tf_all = tfb_ref[...]

        out = jnp.zeros((D_BLK, TFEAT), jnp.float32)
        for h in range(HEADS):
            qh = q_ref[h, pl.ds(d0, D_BLK), :]
            s = _mm(qh, kt_ref[h])
            t = jnp.maximum(s, 0.2 * s) + mask_add
            m = jnp.max(t, axis=1, keepdims=True)
            p = jnp.exp((t - m).astype(jnp.bfloat16))
            ps = p
            for w in (N_SRC // 2, N_SRC // 4, N_SRC // 8):
                ps = ps[:, :w] + ps[:, w:]
            l = jnp.sum(ps.astype(jnp.float32), axis=1, keepdims=True)
            hp = _mm(p, tf_all) * (1.0 / l)
            out = out + jnp.where(hp > 0.0, hp, jnp.exp(hp) - 1.0)
        out_ref[...] = out * (1.0 / HEADS)

        tfb = tfb_ref[pl.ds(d0, D_BLK), :]
        hid = jnp.maximum(_mm(tfb, dw1_ref[...]) + db1_ref[...], 0.0)
        fh_ref[...] = _mm(hid, dw2_ref[...]) + db2_ref[...]


def kernel(bias, emb_dest, emb_src, feature_src, enc_W1, enc_b1, enc_W2,
           enc_b2, dec_W1, dec_b1, dec_W2, dec_b2, att_W, att_W2):
    bf = jnp.bfloat16
    c0 = lambda i: (0, 0)
    tf, feature_src_re, feature_hat = pl.pallas_call(
        _fused_kernel,
        grid=(N_ENC + N_ATT,),
        in_specs=[
            pl.BlockSpec((E_BLK, FEAT), lambda i: (jnp.minimum(i, N_ENC - 1), 0)),
            pl.BlockSpec((FEAT, 2 * TFEAT), c0),
            pl.BlockSpec((1, 2 * TFEAT), c0),
            pl.BlockSpec((2 * TFEAT, TFEAT), c0),
            pl.BlockSpec((1, TFEAT), c0),
            pl.BlockSpec((N_DEST, EMB), c0),
            pl.BlockSpec((N_SRC, EMB), c0),
            pl.BlockSpec((HEADS, EMB, HID), lambda i: (0, 0, 0)),
            pl.BlockSpec((HEADS, HID, HID), lambda i: (0, 0, 0)),
            pl.BlockSpec((D_BLK, N_SRC),
                         lambda i: (jnp.maximum(i - N_ENC, 0), 0)),
            pl.BlockSpec((TFEAT, 2 * TFEAT), c0),
            pl.BlockSpec((1, 2 * TFEAT), c0),
            pl.BlockSpec((2 * TFEAT, FEAT), c0),
            pl.BlockSpec((1, FEAT), c0),
        ],
        out_specs=[
            pl.BlockSpec((E_BLK, TFEAT), lambda i: (jnp.minimum(i, N_ENC - 1), 0)),
            pl.BlockSpec((D_BLK, TFEAT),
                         lambda i: (jnp.maximum(i - N_ENC, 0), 0)),
            pl.BlockSpec((D_BLK, FEAT),
                         lambda i: (jnp.maximum(i - N_ENC, 0), 0)),
        ],
        out_shape=[
            jax.ShapeDtypeStruct((N_SRC, TFEAT), jnp.float32),
            jax.ShapeDtypeStruct((N_DEST, TFEAT), jnp.float32),
            jax.ShapeDtypeStruct((N_SRC, FEAT), jnp.float32),
        ],
        scratch_shapes=[
            pltpu.VMEM((N_SRC, TFEAT), jnp.bfloat16),
            pltpu.VMEM((HEADS, N_DEST, HID), jnp.bfloat16),
            pltpu.VMEM((HEADS, HID, N_SRC), jnp.bfloat16),
        ],
        compiler_params=pltpu.CompilerParams(
            dimension_semantics=("arbitrary",),
        ),
    )(feature_src, enc_W1.astype(bf), enc_b1.reshape(1, -1),
      enc_W2.astype(bf), enc_b2.reshape(1, -1),
      emb_dest, emb_src, att_W.astype(bf), att_W2.astype(bf),
      bias, dec_W1.astype(bf), dec_b1.reshape(1, -1),
      dec_W2.astype(bf), dec_b2.reshape(1, -1))
    return (feature_src_re, feature_hat, tf)
